# baseline (speedup 1.0000x reference)
import jax
import jax.numpy as jnp
from jax.experimental import pallas as pl
from jax.experimental.pallas import tpu as pltpu

_N = 10000
_D = 128
_ALPHA = 0.2
_BM = 400


def _support_body(x_ref, w_ref, o_ref):
    o_ref[...] = jnp.dot(x_ref[...], w_ref[...],
                         preferred_element_type=jnp.float32).astype(jnp.bfloat16)


def _support(x, w):
    return pl.pallas_call(
        _support_body,
        out_shape=jax.ShapeDtypeStruct((_N, _D), jnp.bfloat16),
    )(x, w)


def _leaky(z):
    return jnp.where(z >= 0, z, _ALPHA * z)


def _agg_conv_body(adj_ref, s_ref, b_ref, w2_ref, o_ref, adj16_ref):
    a16 = adj_ref[...].astype(jnp.bfloat16)
    adj16_ref[...] = a16
    z = jnp.dot(a16, s_ref[...], preferred_element_type=jnp.float32) + b_ref[...]
    o_ref[...] = jnp.dot(_leaky(z), w2_ref[...],
                         preferred_element_type=jnp.float32).astype(jnp.bfloat16)


def _agg_conv(adj, s, b, w2):
    return pl.pallas_call(
        _agg_conv_body,
        grid=(_N // _BM,),
        compiler_params=pltpu.CompilerParams(
            dimension_semantics=("parallel",)),
        in_specs=[
            pl.BlockSpec((_BM, _N), lambda i: (i, 0)),
            pl.BlockSpec((_N, _D), lambda i: (0, 0)),
            pl.BlockSpec((1, _D), lambda i: (0, 0)),
            pl.BlockSpec((_D, _D), lambda i: (0, 0)),
        ],
        out_specs=[
            pl.BlockSpec((_BM, _D), lambda i: (i, 0)),
            pl.BlockSpec((_BM, _N), lambda i: (i, 0)),
        ],
        out_shape=[
            jax.ShapeDtypeStruct((_N, _D), jnp.bfloat16),
            jax.ShapeDtypeStruct((_N, _N), jnp.bfloat16),
        ],
    )(adj, s, b.reshape(1, _D), w2)


def _pair_ab_body(adj_ref, s_ref, b_ref, wa_ref, wb_ref,
                  sa_out_ref, act_ref, sb_out_ref):
    a16 = adj_ref[...].astype(jnp.bfloat16)
    z = jnp.dot(a16, s_ref[...], preferred_element_type=jnp.float32) + b_ref[...]
    za = _leaky(z[:, :_D])
    zb = jnp.maximum(z[:, _D:], 0.0)
    sa_out_ref[...] = jnp.dot(za, wa_ref[...],
                              preferred_element_type=jnp.float32).astype(jnp.bfloat16)
    act_ref[...] = zb
    sb_out_ref[...] = jnp.dot(zb, wb_ref[...],
                              preferred_element_type=jnp.float32).astype(jnp.bfloat16)


def _pair_ab(adj, sa, sb, ba, bb, wa, wb):
    s_cat = jnp.concatenate([sa, sb], axis=1)
    b_cat = jnp.concatenate([ba, bb]).reshape(1, 2 * _D)
    return pl.pallas_call(
        _pair_ab_body,
        grid=(_N // _BM,),
        compiler_params=pltpu.CompilerParams(
            dimension_semantics=("parallel",)),
        in_specs=[
            pl.BlockSpec((_BM, _N), lambda i: (i, 0)),
            pl.BlockSpec((_N, 2 * _D), lambda i: (0, 0)),
            pl.BlockSpec((1, 2 * _D), lambda i: (0, 0)),
            pl.BlockSpec((_D, _D), lambda i: (0, 0)),
            pl.BlockSpec((_D, _D), lambda i: (0, 0)),
        ],
        out_specs=[
            pl.BlockSpec((_BM, _D), lambda i: (i, 0)),
            pl.BlockSpec((_BM, _D), lambda i: (i, 0)),
            pl.BlockSpec((_BM, _D), lambda i: (i, 0)),
        ],
        out_shape=[
            jax.ShapeDtypeStruct((_N, _D), jnp.bfloat16),
            jax.ShapeDtypeStruct((_N, _D), jnp.float32),
            jax.ShapeDtypeStruct((_N, _D), jnp.bfloat16),
        ],
    )(adj, s_cat, b_cat, wa, wb)


def _pair_ab_conv_body(adj_ref, s_ref, b_ref, wa_ref, wb_ref,
                       sa_out_ref, act_ref, sb_out_ref, adj16_ref):
    a16 = adj_ref[...].astype(jnp.bfloat16)
    adj16_ref[...] = a16
    z = jnp.dot(a16, s_ref[...], preferred_element_type=jnp.float32) + b_ref[...]
    za = _leaky(z[:, :_D])
    zb = jnp.maximum(z[:, _D:], 0.0)
    sa_out_ref[...] = jnp.dot(za, wa_ref[...],
                              preferred_element_type=jnp.float32).astype(jnp.bfloat16)
    act_ref[...] = zb
    sb_out_ref[...] = jnp.dot(zb, wb_ref[...],
                              preferred_element_type=jnp.float32).astype(jnp.bfloat16)


def _pair_ab_conv(adj, sa, sb, ba, bb, wa, wb):
    s_cat = jnp.concatenate([sa, sb], axis=1)
    b_cat = jnp.concatenate([ba, bb]).reshape(1, 2 * _D)
    return pl.pallas_call(
        _pair_ab_conv_body,
        grid=(_N // _BM,),
        compiler_params=pltpu.CompilerParams(
            dimension_semantics=("parallel",)),
        in_specs=[
            pl.BlockSpec((_BM, _N), lambda i: (i, 0)),
            pl.BlockSpec((_N, 2 * _D), lambda i: (0, 0)),
            pl.BlockSpec((1, 2 * _D), lambda i: (0, 0)),
            pl.BlockSpec((_D, _D), lambda i: (0, 0)),
            pl.BlockSpec((_D, _D), lambda i: (0, 0)),
        ],
        out_specs=[
            pl.BlockSpec((_BM, _D), lambda i: (i, 0)),
            pl.BlockSpec((_BM, _D), lambda i: (i, 0)),
            pl.BlockSpec((_BM, _D), lambda i: (i, 0)),
            pl.BlockSpec((_BM, _N), lambda i: (i, 0)),
        ],
        out_shape=[
            jax.ShapeDtypeStruct((_N, _D), jnp.bfloat16),
            jax.ShapeDtypeStruct((_N, _D), jnp.float32),
            jax.ShapeDtypeStruct((_N, _D), jnp.bfloat16),
            jax.ShapeDtypeStruct((_N, _N), jnp.bfloat16),
        ],
    )(adj, s_cat, b_cat, wa, wb)


def _pair_amean_body(adj_ref, s_ref, b_ref, wa_ref, x0_ref, x1_ref,
                     sa_out_ref, mean_ref):
    a16 = adj_ref[...].astype(jnp.bfloat16)
    z = jnp.dot(a16, s_ref[...], preferred_element_type=jnp.float32) + b_ref[...]
    za = _leaky(z[:, :_D])
    zb = jnp.maximum(z[:, _D:], 0.0)
    sa_out_ref[...] = jnp.dot(za, wa_ref[...],
                              preferred_element_type=jnp.float32).astype(jnp.bfloat16)
    mean_ref[...] = (x0_ref[...] + x1_ref[...] + zb) * (1.0 / 3.0)


def _pair_amean(adj, sa, sb, ba, bb, wa, x0, x1):
    s_cat = jnp.concatenate([sa, sb], axis=1)
    b_cat = jnp.concatenate([ba, bb]).reshape(1, 2 * _D)
    return pl.pallas_call(
        _pair_amean_body,
        grid=(_N // _BM,),
        compiler_params=pltpu.CompilerParams(
            dimension_semantics=("parallel",)),
        in_specs=[
            pl.BlockSpec((_BM, _N), lambda i: (i, 0)),
            pl.BlockSpec((_N, 2 * _D), lambda i: (0, 0)),
            pl.BlockSpec((1, 2 * _D), lambda i: (0, 0)),
            pl.BlockSpec((_D, _D), lambda i: (0, 0)),
            pl.BlockSpec((_BM, _D), lambda i: (i, 0)),
            pl.BlockSpec((_BM, _D), lambda i: (i, 0)),
        ],
        out_specs=[
            pl.BlockSpec((_BM, _D), lambda i: (i, 0)),
            pl.BlockSpec((_BM, _D), lambda i: (i, 0)),
        ],
        out_shape=[
            jax.ShapeDtypeStruct((_N, _D), jnp.bfloat16),
            jax.ShapeDtypeStruct((_N, _D), jnp.float32),
        ],
    )(adj, s_cat, b_cat, wa, x0, x1)


def _agg_mean_body(adj_ref, s_ref, b_ref, x0_ref, x1_ref, o_ref):
    z = jnp.dot(adj_ref[...], s_ref[...],
                preferred_element_type=jnp.float32) + b_ref[...]
    z = jnp.maximum(z, 0.0)
    o_ref[...] = (x0_ref[...] + x1_ref[...] + z) * (1.0 / 3.0)


def _agg_mean(adj, s, b, x0, x1):
    return pl.pallas_call(
        _agg_mean_body,
        grid=(_N // _BM,),
        compiler_params=pltpu.CompilerParams(
            dimension_semantics=("parallel",)),
        in_specs=[
            pl.BlockSpec((_BM, _N), lambda i: (i, 0)),
            pl.BlockSpec((_N, _D), lambda i: (0, 0)),
            pl.BlockSpec((1, _D), lambda i: (0, 0)),
            pl.BlockSpec((_BM, _D), lambda i: (i, 0)),
            pl.BlockSpec((_BM, _D), lambda i: (i, 0)),
        ],
        out_specs=pl.BlockSpec((_BM, _D), lambda i: (i, 0)),
        out_shape=jax.ShapeDtypeStruct((_N, _D), jnp.float32),
    )(adj, s, b.reshape(1, _D), x0, x1)


def kernel(ufea, vfea, UV_adj, VU_adj, params):
    p = params
    s_u0 = _support(ufea, p["W_0_0"])
    s_v0 = _support(vfea, p["W_0_1"])
    s_u1, VU16 = _agg_conv(VU_adj, s_u0, p["b_0_0"], p["W_0_2"])
    s_v1, u1, s_u_l1, UV16 = _pair_ab_conv(UV_adj, s_v0, s_u1,
                                p["b_0_1"], p["b_0_2"], p["W_0_3"], p["W_1_0"])
    s_u1p, v1, s_v_l1 = _pair_ab(VU16, s_u_l1, s_v1,
                                 p["b_1_0"], p["b_0_3"], p["W_1_2"], p["W_1_1"])
    s_v1p, out_u = _pair_amean(UV16, s_v_l1, s_u1p,
                               p["b_1_1"], p["b_1_2"], p["W_1_3"], ufea, u1)
    out_v = _agg_mean(VU16, s_v1p, p["b_1_3"], vfea, v1)
    return (out_u, out_v)


# speedup vs baseline: 1.0530x; 1.0530x over previous
import jax
import jax.numpy as jnp
from jax.experimental import pallas as pl
from jax.experimental.pallas import tpu as pltpu

_N = 10000
_D = 128
_ALPHA = 0.2
_BM = 400
_PAR = pltpu.CompilerParams(dimension_semantics=("parallel",))


def _leaky(z):
    return jnp.where(z >= 0, z, _ALPHA * z)


def _p1_body(adj_ref, ufea_ref, vfea_ref, b_ref, w00_ref, w01_ref, w02_ref,
             scat_ref, adj16_ref, su0_ref):
    @pl.when(pl.program_id(0) == 0)
    def _():
        su0_ref[...] = jnp.dot(ufea_ref[...].astype(jnp.bfloat16), w00_ref[...],
                               preferred_element_type=jnp.float32).astype(jnp.bfloat16)

    a16 = adj_ref[...].astype(jnp.bfloat16)
    adj16_ref[...] = a16
    z = jnp.dot(a16, su0_ref[...], preferred_element_type=jnp.float32) + b_ref[...]
    scat_ref[:, :_D] = jnp.dot(vfea_ref[...].astype(jnp.bfloat16), w01_ref[...],
                               preferred_element_type=jnp.float32).astype(jnp.bfloat16)
    scat_ref[:, _D:] = jnp.dot(_leaky(z), w02_ref[...],
                               preferred_element_type=jnp.float32).astype(jnp.bfloat16)


def _p1(adj, ufea, vfea, b00, w00, w01, w02):
    return pl.pallas_call(
        _p1_body,
        grid=(_N // _BM,),
        compiler_params=_PAR,
        in_specs=[
            pl.BlockSpec((_BM, _N), lambda i: (i, 0)),
            pl.BlockSpec((_N, _D), lambda i: (0, 0)),
            pl.BlockSpec((_BM, _D), lambda i: (i, 0)),
            pl.BlockSpec((1, _D), lambda i: (0, 0)),
            pl.BlockSpec((_D, _D), lambda i: (0, 0)),
            pl.BlockSpec((_D, _D), lambda i: (0, 0)),
            pl.BlockSpec((_D, _D), lambda i: (0, 0)),
        ],
        out_specs=[
            pl.BlockSpec((_BM, 2 * _D), lambda i: (i, 0)),
            pl.BlockSpec((_BM, _N), lambda i: (i, 0)),
        ],
        out_shape=[
            jax.ShapeDtypeStruct((_N, 2 * _D), jnp.bfloat16),
            jax.ShapeDtypeStruct((_N, _N), jnp.bfloat16),
        ],
        scratch_shapes=[pltpu.VMEM((_N, _D), jnp.bfloat16)],
    )(adj, ufea, vfea, b00.reshape(1, _D), w00, w01, w02)


def _pair_ab_body(adj_ref, s_ref, b_ref, wa_ref, wb_ref, scat_ref, act_ref):
    a16 = adj_ref[...].astype(jnp.bfloat16)
    z = jnp.dot(a16, s_ref[...], preferred_element_type=jnp.float32) + b_ref[...]
    za = _leaky(z[:, :_D])
    zb = jnp.maximum(z[:, _D:], 0.0)
    scat_ref[:, :_D] = jnp.dot(zb, wb_ref[...],
                               preferred_element_type=jnp.float32).astype(jnp.bfloat16)
    scat_ref[:, _D:] = jnp.dot(za, wa_ref[...],
                               preferred_element_type=jnp.float32).astype(jnp.bfloat16)
    act_ref[...] = zb


def _pair_ab(adj, s_cat, ba, bb, wa, wb):
    b_cat = jnp.concatenate([ba, bb]).reshape(1, 2 * _D)
    return pl.pallas_call(
        _pair_ab_body,
        grid=(_N // _BM,),
        compiler_params=_PAR,
        in_specs=[
            pl.BlockSpec((_BM, _N), lambda i: (i, 0)),
            pl.BlockSpec((_N, 2 * _D), lambda i: (0, 0)),
            pl.BlockSpec((1, 2 * _D), lambda i: (0, 0)),
            pl.BlockSpec((_D, _D), lambda i: (0, 0)),
            pl.BlockSpec((_D, _D), lambda i: (0, 0)),
        ],
        out_specs=[
            pl.BlockSpec((_BM, 2 * _D), lambda i: (i, 0)),
            pl.BlockSpec((_BM, _D), lambda i: (i, 0)),
        ],
        out_shape=[
            jax.ShapeDtypeStruct((_N, 2 * _D), jnp.bfloat16),
            jax.ShapeDtypeStruct((_N, _D), jnp.float32),
        ],
    )(adj, s_cat, b_cat, wa, wb)


def _pair_amean_body(adj_ref, s_ref, b_ref, wa_ref, x0_ref, x1_ref,
                     sa_out_ref, mean_ref):
    a16 = adj_ref[...].astype(jnp.bfloat16)
    z = jnp.dot(a16, s_ref[...], preferred_element_type=jnp.float32) + b_ref[...]
    za = _leaky(z[:, :_D])
    zb = jnp.maximum(z[:, _D:], 0.0)
    sa_out_ref[...] = jnp.dot(za, wa_ref[...],
                              preferred_element_type=jnp.float32).astype(jnp.bfloat16)
    mean_ref[...] = (x0_ref[...] + x1_ref[...] + zb) * (1.0 / 3.0)


def _pair_amean(adj, s_cat, ba, bb, wa, x0, x1):
    b_cat = jnp.concatenate([ba, bb]).reshape(1, 2 * _D)
    return pl.pallas_call(
        _pair_amean_body,
        grid=(_N // _BM,),
        compiler_params=_PAR,
        in_specs=[
            pl.BlockSpec((_BM, _N), lambda i: (i, 0)),
            pl.BlockSpec((_N, 2 * _D), lambda i: (0, 0)),
            pl.BlockSpec((1, 2 * _D), lambda i: (0, 0)),
            pl.BlockSpec((_D, _D), lambda i: (0, 0)),
            pl.BlockSpec((_BM, _D), lambda i: (i, 0)),
            pl.BlockSpec((_BM, _D), lambda i: (i, 0)),
        ],
        out_specs=[
            pl.BlockSpec((_BM, _D), lambda i: (i, 0)),
            pl.BlockSpec((_BM, _D), lambda i: (i, 0)),
        ],
        out_shape=[
            jax.ShapeDtypeStruct((_N, _D), jnp.bfloat16),
            jax.ShapeDtypeStruct((_N, _D), jnp.float32),
        ],
    )(adj, s_cat, b_cat, wa, x0, x1)


def _agg_mean_body(adj_ref, s_ref, b_ref, x0_ref, x1_ref, o_ref):
    z = jnp.dot(adj_ref[...], s_ref[...],
                preferred_element_type=jnp.float32) + b_ref[...]
    z = jnp.maximum(z, 0.0)
    o_ref[...] = (x0_ref[...] + x1_ref[...] + z) * (1.0 / 3.0)


def _agg_mean(adj, s, b, x0, x1):
    return pl.pallas_call(
        _agg_mean_body,
        grid=(_N // _BM,),
        compiler_params=_PAR,
        in_specs=[
            pl.BlockSpec((_BM, _N), lambda i: (i, 0)),
            pl.BlockSpec((_N, _D), lambda i: (0, 0)),
            pl.BlockSpec((1, _D), lambda i: (0, 0)),
            pl.BlockSpec((_BM, _D), lambda i: (i, 0)),
            pl.BlockSpec((_BM, _D), lambda i: (i, 0)),
        ],
        out_specs=pl.BlockSpec((_BM, _D), lambda i: (i, 0)),
        out_shape=jax.ShapeDtypeStruct((_N, _D), jnp.float32),
    )(adj, s, b.reshape(1, _D), x0, x1)


def kernel(ufea, vfea, UV_adj, VU_adj, params):
    p = params
    s_cat, VU16 = _p1(VU_adj, ufea, vfea,
                      p["b_0_0"], p["W_0_0"], p["W_0_1"], p["W_0_2"])
    s_cat, u1 = _pair_ab(UV_adj, s_cat,
                         p["b_0_1"], p["b_0_2"], p["W_0_3"], p["W_1_0"])
    s_cat, v1 = _pair_ab(VU16, s_cat,
                         p["b_1_0"], p["b_0_3"], p["W_1_2"], p["W_1_1"])
    s_v1p, out_u = _pair_amean(UV_adj, s_cat,
                               p["b_1_1"], p["b_1_2"], p["W_1_3"], ufea, u1)
    out_v = _agg_mean(VU16, s_v1p, p["b_1_3"], vfea, v1)
    return (out_u, out_v)


# speedup vs baseline: 1.0677x; 1.0140x over previous
import jax
import jax.numpy as jnp
from jax.experimental import pallas as pl
from jax.experimental.pallas import tpu as pltpu

_N = 10000
_D = 128
_ALPHA = 0.2
_BM = 400
_S = _N // _BM
_VMEM = pltpu.CompilerParams(
    dimension_semantics=("arbitrary",),
    vmem_limit_bytes=63 * 1024 * 1024,
)


def _leaky(z):
    return jnp.where(z >= 0, z, _ALPHA * z)


def _bf(x):
    return x.astype(jnp.bfloat16)


def _dot(a, b):
    return jnp.dot(a, b, preferred_element_type=jnp.float32)


def _p1_body(adj_ref, ufea_ref, vfea_ref, b_ref, w00_ref, w01_ref, w02_ref,
             scat_ref, adj16_ref, su0_ref):
    @pl.when(pl.program_id(0) == 0)
    def _():
        su0_ref[...] = _bf(_dot(_bf(ufea_ref[...]), w00_ref[...]))

    a16 = _bf(adj_ref[...])
    adj16_ref[...] = a16
    z = _dot(a16, su0_ref[...]) + b_ref[...]
    scat_ref[:, :_D] = _bf(_dot(_bf(vfea_ref[...]), w01_ref[...]))
    scat_ref[:, _D:] = _bf(_dot(_leaky(z), w02_ref[...]))


def _p1(adj, ufea, vfea, b00, w00, w01, w02):
    return pl.pallas_call(
        _p1_body,
        grid=(_S,),
        compiler_params=_VMEM,
        in_specs=[
            pl.BlockSpec((_BM, _N), lambda i: (i, 0)),
            pl.BlockSpec((_N, _D), lambda i: (0, 0)),
            pl.BlockSpec((_BM, _D), lambda i: (i, 0)),
            pl.BlockSpec((1, _D), lambda i: (0, 0)),
            pl.BlockSpec((_D, _D), lambda i: (0, 0)),
            pl.BlockSpec((_D, _D), lambda i: (0, 0)),
            pl.BlockSpec((_D, _D), lambda i: (0, 0)),
        ],
        out_specs=[
            pl.BlockSpec((_BM, 2 * _D), lambda i: (i, 0)),
            pl.BlockSpec((_BM, _N), lambda i: (i, 0)),
        ],
        out_shape=[
            jax.ShapeDtypeStruct((_N, 2 * _D), jnp.bfloat16),
            jax.ShapeDtypeStruct((_N, _N), jnp.bfloat16),
        ],
        scratch_shapes=[pltpu.VMEM((_N, _D), jnp.bfloat16)],
    )(adj, ufea, vfea, b00.reshape(1, _D), w00, w01, w02)


def _mid_body(uv_ref, vu16_ref, scat_in_ref, ufea_ref, vfea_ref,
              bcat1_ref, bcat2_ref, w03_ref, w10_ref, w11_ref, w12_ref,
              xu01_ref, scat_out_ref, xv01_ref, scat_scr):
    k = pl.program_id(0)

    @pl.when(k < _S)
    def _():
        a16 = _bf(uv_ref[...])
        z = _dot(a16, scat_in_ref[...]) + bcat1_ref[...]
        za = _leaky(z[:, :_D])
        zb = jnp.maximum(z[:, _D:], 0.0)
        base = k * _BM
        scat_scr[pl.ds(base, _BM), :_D] = _bf(_dot(zb, w10_ref[...]))
        scat_scr[pl.ds(base, _BM), _D:] = _bf(_dot(za, w03_ref[...]))
        xu01_ref[...] = ufea_ref[...] + zb

    @pl.when(k >= _S)
    def _():
        z = _dot(vu16_ref[...], scat_scr[...]) + bcat2_ref[...]
        za = _leaky(z[:, :_D])
        zb = jnp.maximum(z[:, _D:], 0.0)
        scat_out_ref[:, :_D] = _bf(_dot(zb, w11_ref[...]))
        scat_out_ref[:, _D:] = _bf(_dot(za, w12_ref[...]))
        xv01_ref[...] = vfea_ref[...] + zb


def _mid(UV_adj, VU16, s_cat, ufea, vfea, p):
    bcat1 = jnp.concatenate([p["b_0_1"], p["b_0_2"]]).reshape(1, 2 * _D)
    bcat2 = jnp.concatenate([p["b_1_0"], p["b_0_3"]]).reshape(1, 2 * _D)
    i_a = lambda k: (jnp.minimum(k, _S - 1), 0)
    i_b = lambda k: (jnp.maximum(k, _S) - _S, 0)
    return pl.pallas_call(
        _mid_body,
        grid=(2 * _S,),
        compiler_params=_VMEM,
        in_specs=[
            pl.BlockSpec((_BM, _N), i_a),
            pl.BlockSpec((_BM, _N), i_b),
            pl.BlockSpec((_N, 2 * _D), lambda k: (0, 0)),
            pl.BlockSpec((_BM, _D), i_a),
            pl.BlockSpec((_BM, _D), i_b),
            pl.BlockSpec((1, 2 * _D), lambda k: (0, 0)),
            pl.BlockSpec((1, 2 * _D), lambda k: (0, 0)),
            pl.BlockSpec((_D, _D), lambda k: (0, 0)),
            pl.BlockSpec((_D, _D), lambda k: (0, 0)),
            pl.BlockSpec((_D, _D), lambda k: (0, 0)),
            pl.BlockSpec((_D, _D), lambda k: (0, 0)),
        ],
        out_specs=[
            pl.BlockSpec((_BM, _D), i_a),
            pl.BlockSpec((_BM, 2 * _D), i_b),
            pl.BlockSpec((_BM, _D), i_b),
        ],
        out_shape=[
            jax.ShapeDtypeStruct((_N, _D), jnp.float32),
            jax.ShapeDtypeStruct((_N, 2 * _D), jnp.bfloat16),
            jax.ShapeDtypeStruct((_N, _D), jnp.float32),
        ],
        scratch_shapes=[pltpu.VMEM((_N, 2 * _D), jnp.bfloat16)],
    )(UV_adj, VU16, s_cat, ufea, vfea, bcat1, bcat2,
      p["W_0_3"], p["W_1_0"], p["W_1_1"], p["W_1_2"])


def _last_body(uv_ref, vu16_ref, scat_in_ref, xu01_ref, xv01_ref,
               bcat3_ref, b13_ref, w13_ref, outu_ref, outv_ref, svp_scr):
    k = pl.program_id(0)

    @pl.when(k < _S)
    def _():
        a16 = _bf(uv_ref[...])
        z = _dot(a16, scat_in_ref[...]) + bcat3_ref[...]
        za = _leaky(z[:, :_D])
        zb = jnp.maximum(z[:, _D:], 0.0)
        base = k * _BM
        svp_scr[pl.ds(base, _BM), :] = _bf(_dot(za, w13_ref[...]))
        outu_ref[...] = (xu01_ref[...] + zb) * (1.0 / 3.0)

    @pl.when(k >= _S)
    def _():
        z = _dot(vu16_ref[...], svp_scr[...]) + b13_ref[...]
        zb = jnp.maximum(z, 0.0)
        outv_ref[...] = (xv01_ref[...] + zb) * (1.0 / 3.0)


def _last(UV_adj, VU16, s_cat, xu01, xv01, p):
    bcat3 = jnp.concatenate([p["b_1_1"], p["b_1_2"]]).reshape(1, 2 * _D)
    i_a = lambda k: (jnp.minimum(k, _S - 1), 0)
    i_b = lambda k: (jnp.maximum(k, _S) - _S, 0)
    return pl.pallas_call(
        _last_body,
        grid=(2 * _S,),
        compiler_params=_VMEM,
        in_specs=[
            pl.BlockSpec((_BM, _N), i_a),
            pl.BlockSpec((_BM, _N), i_b),
            pl.BlockSpec((_N, 2 * _D), lambda k: (0, 0)),
            pl.BlockSpec((_BM, _D), i_a),
            pl.BlockSpec((_BM, _D), i_b),
            pl.BlockSpec((1, 2 * _D), lambda k: (0, 0)),
            pl.BlockSpec((1, _D), lambda k: (0, 0)),
            pl.BlockSpec((_D, _D), lambda k: (0, 0)),
        ],
        out_specs=[
            pl.BlockSpec((_BM, _D), i_a),
            pl.BlockSpec((_BM, _D), i_b),
        ],
        out_shape=[
            jax.ShapeDtypeStruct((_N, _D), jnp.float32),
            jax.ShapeDtypeStruct((_N, _D), jnp.float32),
        ],
        scratch_shapes=[pltpu.VMEM((_N, _D), jnp.bfloat16)],
    )(UV_adj, VU16, s_cat, xu01, xv01, bcat3,
      p["b_1_3"].reshape(1, _D), p["W_1_3"])


def kernel(ufea, vfea, UV_adj, VU_adj, params):
    p = params
    s_cat, VU16 = _p1(VU_adj, ufea, vfea,
                      p["b_0_0"], p["W_0_0"], p["W_0_1"], p["W_0_2"])
    xu01, s_cat, xv01 = _mid(UV_adj, VU16, s_cat, ufea, vfea, p)
    out_u, out_v = _last(UV_adj, VU16, s_cat, xu01, xv01, p)
    return (out_u, out_v)
